# Initial kernel scaffold; baseline (speedup 1.0000x reference)
import jax
import jax.numpy as jnp
from jax.experimental import pallas as pl


def kernel(x, mean, std, W_enc, b_enc, memory, mem_data, count):
    raise NotImplementedError("write your pallas kernel here")


# speedup vs baseline: 1.3276x; 1.3276x over previous
import functools

import jax
import jax.numpy as jnp
from jax.experimental import pallas as pl
from jax.experimental.pallas import tpu as pltpu

IN_DIM = 128
OUT_DIM = 256
MEM_LEN = 100000
BETA = 2000.0

BLK = 2000
NBLK = MEM_LEN // BLK


def _body(pos_ref, x_ref, mean_ref, std_ref, w_ref, b_ref, mem_ref, md_ref,
          loss_ref, out_mem_ref, out_md_ref, enc_ref, min_ref):
    i = pl.program_id(0)

    @pl.when(i == 0)
    def _encode():
        xv = x_ref[...]
        std = std_ref[...]
        new = jnp.where(std == 0.0, 0.0, (xv - mean_ref[...]) / std)
        logits = jnp.dot(new, w_ref[...],
                         preferred_element_type=jnp.float32) + b_ref[...]
        m = jnp.max(logits)
        lse = jnp.log(jnp.sum(jnp.exp(logits - m))) + m
        enc_ref[...] = logits - lse
        min_ref[0] = jnp.inf

    @pl.when(i < NBLK)
    def _stream():
        blk = mem_ref[...]
        out_mem_ref[...] = blk
        out_md_ref[...] = md_ref[...]
        d = jnp.sum(jnp.abs(blk - enc_ref[...]), axis=1)
        min_ref[0] = jnp.minimum(min_ref[0], jnp.min(d))

    @pl.when(i == NBLK)
    def _fixup():
        loss = min_ref[0]
        loss_ref[...] = jnp.full((1, 1), loss, jnp.float32)
        do_update = loss <= BETA
        r = pos_ref[0] % BLK
        row_sel = jax.lax.broadcasted_iota(jnp.int32, (BLK, 1), 0) == r
        sel = jnp.logical_and(do_update, row_sel)
        out_mem_ref[...] = jnp.where(sel, enc_ref[...], mem_ref[...])
        out_md_ref[...] = jnp.where(sel, x_ref[...], md_ref[...])


def kernel(x, mean, std, W_enc, b_enc, memory, mem_data, count):
    pos = jnp.asarray(count % MEM_LEN, jnp.int32).reshape(1)
    mean2 = mean.reshape(1, IN_DIM)
    std2 = std.reshape(1, IN_DIM)
    b2 = b_enc.reshape(1, OUT_DIM)

    def big_map(i, p):
        return (jnp.where(i < NBLK, i, p[0] // BLK), 0)

    def const_map(i, p):
        return (0, 0)

    grid_spec = pltpu.PrefetchScalarGridSpec(
        num_scalar_prefetch=1,
        grid=(NBLK + 1,),
        in_specs=[
            pl.BlockSpec((1, IN_DIM), const_map),
            pl.BlockSpec((1, IN_DIM), const_map),
            pl.BlockSpec((1, IN_DIM), const_map),
            pl.BlockSpec((IN_DIM, OUT_DIM), const_map),
            pl.BlockSpec((1, OUT_DIM), const_map),
            pl.BlockSpec((BLK, OUT_DIM), big_map),
            pl.BlockSpec((BLK, IN_DIM), big_map),
        ],
        out_specs=[
            pl.BlockSpec((1, 1), const_map),
            pl.BlockSpec((BLK, OUT_DIM), big_map),
            pl.BlockSpec((BLK, IN_DIM), big_map),
        ],
        scratch_shapes=[
            pltpu.VMEM((1, OUT_DIM), jnp.float32),
            pltpu.SMEM((1,), jnp.float32),
        ],
    )

    loss2d, new_memory, new_mem_data = pl.pallas_call(
        _body,
        grid_spec=grid_spec,
        out_shape=[
            jax.ShapeDtypeStruct((1, 1), jnp.float32),
            jax.ShapeDtypeStruct((MEM_LEN, OUT_DIM), jnp.float32),
            jax.ShapeDtypeStruct((MEM_LEN, IN_DIM), jnp.float32),
        ],
        compiler_params=pltpu.CompilerParams(
            dimension_semantics=("arbitrary",),
        ),
    )(pos, x, mean2, std2, W_enc, b2, memory, mem_data)

    return loss2d.reshape(()), new_memory, new_mem_data


# speedup vs baseline: 1.3519x; 1.0183x over previous
import functools

import jax
import jax.numpy as jnp
from jax.experimental import pallas as pl
from jax.experimental.pallas import tpu as pltpu

IN_DIM = 128
OUT_DIM = 256
MEM_LEN = 100000
BETA = 2000.0

BLK = 5000
NBLK = MEM_LEN // BLK


def _body(pos_ref, x_ref, mean_ref, std_ref, w_ref, b_ref, mem_ref, md_ref,
          loss_ref, out_mem_ref, out_md_ref, enc_ref, min_ref):
    i = pl.program_id(0)

    @pl.when(i == 0)
    def _encode():
        xv = x_ref[...]
        std = std_ref[...]
        new = jnp.where(std == 0.0, 0.0, (xv - mean_ref[...]) / std)
        logits = jnp.dot(new, w_ref[...],
                         preferred_element_type=jnp.float32) + b_ref[...]
        m = jnp.max(logits)
        lse = jnp.log(jnp.sum(jnp.exp(logits - m))) + m
        enc_ref[...] = logits - lse
        min_ref[0] = jnp.inf

    @pl.when(i < NBLK)
    def _stream():
        blk = mem_ref[...]
        out_mem_ref[...] = blk
        out_md_ref[...] = md_ref[...]
        d = jnp.sum(jnp.abs(blk - enc_ref[...]), axis=1)
        min_ref[0] = jnp.minimum(min_ref[0], jnp.min(d))

    @pl.when(i == NBLK)
    def _fixup():
        loss = min_ref[0]
        loss_ref[...] = jnp.full((1, 1), loss, jnp.float32)
        do_update = loss <= BETA
        r = pos_ref[0] % BLK
        row_sel = jax.lax.broadcasted_iota(jnp.int32, (BLK, 1), 0) == r
        sel = jnp.logical_and(do_update, row_sel)
        out_mem_ref[...] = jnp.where(sel, enc_ref[...], mem_ref[...])
        out_md_ref[...] = jnp.where(sel, x_ref[...], md_ref[...])


def kernel(x, mean, std, W_enc, b_enc, memory, mem_data, count):
    pos = jnp.asarray(count % MEM_LEN, jnp.int32).reshape(1)
    mean2 = mean.reshape(1, IN_DIM)
    std2 = std.reshape(1, IN_DIM)
    b2 = b_enc.reshape(1, OUT_DIM)

    def big_map(i, p):
        return (jnp.where(i < NBLK, i, p[0] // BLK), 0)

    def const_map(i, p):
        return (0, 0)

    grid_spec = pltpu.PrefetchScalarGridSpec(
        num_scalar_prefetch=1,
        grid=(NBLK + 1,),
        in_specs=[
            pl.BlockSpec((1, IN_DIM), const_map),
            pl.BlockSpec((1, IN_DIM), const_map),
            pl.BlockSpec((1, IN_DIM), const_map),
            pl.BlockSpec((IN_DIM, OUT_DIM), const_map),
            pl.BlockSpec((1, OUT_DIM), const_map),
            pl.BlockSpec((BLK, OUT_DIM), big_map),
            pl.BlockSpec((BLK, IN_DIM), big_map),
        ],
        out_specs=[
            pl.BlockSpec((1, 1), const_map),
            pl.BlockSpec((BLK, OUT_DIM), big_map),
            pl.BlockSpec((BLK, IN_DIM), big_map),
        ],
        scratch_shapes=[
            pltpu.VMEM((1, OUT_DIM), jnp.float32),
            pltpu.SMEM((1,), jnp.float32),
        ],
    )

    loss2d, new_memory, new_mem_data = pl.pallas_call(
        _body,
        grid_spec=grid_spec,
        out_shape=[
            jax.ShapeDtypeStruct((1, 1), jnp.float32),
            jax.ShapeDtypeStruct((MEM_LEN, OUT_DIM), jnp.float32),
            jax.ShapeDtypeStruct((MEM_LEN, IN_DIM), jnp.float32),
        ],
        compiler_params=pltpu.CompilerParams(
            dimension_semantics=("arbitrary",),
        ),
    )(pos, x, mean2, std2, W_enc, b2, memory, mem_data)

    return loss2d.reshape(()), new_memory, new_mem_data
